# baseline (speedup 1.0000x reference)
import functools

import jax
import jax.numpy as jnp
from jax import lax
from jax.experimental import pallas as pl
from jax.experimental.pallas import tpu as pltpu
from jax.experimental.pallas import tpu_sc as plsc

B = 16384
NSLICE = 4
BS = B // NSLICE
L = 20
D = 128
NCLS = 732
NCAT = 100
NC = 2
NS = 16
NW = NC * NS
CB = 8
NJ = D // 16

_F32_MAX = 3.4028235e38


def _tree_sum(xs):
    xs = list(xs)
    while len(xs) > 1:
        nxt = [xs[i] + xs[i + 1] for i in range(0, len(xs) - 1, 2)]
        if len(xs) % 2:
            nxt.append(xs[-1])
        xs = nxt
    return xs[0]


def _make_pool(bs, boff):
    per_w = bs // NW
    chunks = per_w // CB
    npair = chunks // 2

    gd = 2
    gr = CB * L // gd

    def body(textt_hbm, emb_hbm, xpool_hbm, tidxt, tidx_all, rows0, rows1,
             out0, out1, gsem0, gsem1, ssem0, ssem1):
        wid = lax.axis_index("s") * NC + lax.axis_index("c")
        wbase = wid * per_w
        pltpu.sync_copy(textt_hbm.at[:, pl.ds(boff + wbase, per_w)], tidxt)

        lanes = lax.iota(jnp.int32, 16)

        def tgroup(g, _):
            base = g * 16
            for r in range(L):
                v = tidxt[r, pl.ds(base, 16)]
                plsc.store_scatter(tidx_all, [lanes * L + (base * L + r)], v)
            return 0

        lax.fori_loop(0, per_w // 16, tgroup, 0)

        rows = (rows0, rows1)
        outs = (out0, out1)
        gsems = (gsem0, gsem1)
        ssems = (ssem0, ssem1)

        def fire(c, slot):
            for h in range(gd):
                pltpu.async_copy(
                    emb_hbm.at[tidx_all.at[pl.ds(c * CB * L + h * gr, gr)]],
                    rows[slot].at[pl.ds(h * gr, gr)], gsems[slot])

        def wait_gathers(c, slot):
            for h in range(gd):
                pltpu.make_async_copy(
                    emb_hbm.at[tidx_all.at[pl.ds(c * CB * L + h * gr, gr)]],
                    rows[slot].at[pl.ds(h * gr, gr)], gsems[slot]).wait()

        def wait_store(c, slot):
            pltpu.make_async_copy(
                outs[slot], xpool_hbm.at[pl.ds(wbase + c * CB, CB)],
                ssems[slot]).wait()

        def compute(c, slot):
            rows_v = rows[slot]
            out_v = outs[slot]

            def elem_body(e, _):
                ebase = e * L
                acc_a = [jnp.zeros((16,), jnp.float32) for _ in range(NJ)]
                acc_b = [jnp.zeros((16,), jnp.float32) for _ in range(NJ)]
                inds = []
                for r in range(L):
                    vs = [rows_v[ebase + r, pl.ds(j * 16, 16)]
                          for j in range(NJ)]
                    if r % 2 == 0:
                        acc_a = [a + v for a, v in zip(acc_a, vs)]
                    else:
                        acc_b = [a + v for a, v in zip(acc_b, vs)]
                    s = jnp.sum(_tree_sum(vs))
                    inds.append(jnp.where(s != 0.0, 1.0, 0.0))
                cnt = _tree_sum(inds)
                invv = 1.0 / (cnt + jnp.zeros((16,), jnp.float32))
                for j in range(NJ):
                    q = (acc_a[j] + acc_b[j]) * invv
                    q = jnp.where(jnp.isnan(q), 0.0, q)
                    q = jnp.where(q == jnp.inf, _F32_MAX, q)
                    q = jnp.where(q == -jnp.inf, -_F32_MAX, q)
                    out_v[e, pl.ds(j * 16, 16)] = q
                return 0

            lax.fori_loop(0, CB, elem_body, 0)
            pltpu.async_copy(out_v, xpool_hbm.at[pl.ds(wbase + c * CB, CB)],
                             ssems[slot])

        fire(0, 0)
        fire(1, 1)

        def pair_body(p, _):
            for slot in range(2):
                c = 2 * p + slot
                wait_gathers(c, slot)

                @pl.when(p > 0)
                def _():
                    wait_store(c - 2, slot)

                compute(c, slot)

                @pl.when(p < npair - 1)
                def _():
                    fire(c + 2, slot)

            return 0

        lax.fori_loop(0, npair, pair_body, 0)
        wait_store(chunks - 2, 0)
        wait_store(chunks - 1, 1)

    return pl.kernel(
        body,
        out_type=jax.ShapeDtypeStruct((bs, D), jnp.float32),
        mesh=plsc.VectorSubcoreMesh(core_axis_name="c", subcore_axis_name="s"),
        compiler_params=pltpu.CompilerParams(needs_layout_passes=False),
        scratch_types=[
            pltpu.VMEM((L, per_w), jnp.int32),
            pltpu.VMEM((per_w * L,), jnp.int32),
            pltpu.VMEM((CB * L, D), jnp.float32),
            pltpu.VMEM((CB * L, D), jnp.float32),
            pltpu.VMEM((CB, D), jnp.float32),
            pltpu.VMEM((CB, D), jnp.float32),
            pltpu.SemaphoreType.DMA,
            pltpu.SemaphoreType.DMA,
            pltpu.SemaphoreType.DMA,
            pltpu.SemaphoreType.DMA,
        ],
    )


_sc_pools = [_make_pool(BS, k * BS) for k in range(NSLICE)]


BM = 1024
NBS = BS // BM


def _head_body_plain(x_ref, i0_ref, i1_ref, i2_ref, cc_ref, w_ref, b_ref,
                     o_ref):
    span = lax.broadcasted_iota(jnp.int32, (BM, 3 * NCAT), 1)
    oh = ((span == i0_ref[0, 0, :][:, None]).astype(jnp.float32)
          + (span == i1_ref[0, 0, :][:, None] + NCAT).astype(jnp.float32)
          + (span == i2_ref[0, 0, :][:, None] + 2 * NCAT).astype(jnp.float32))
    cat = jnp.dot(oh, cc_ref[...], preferred_element_type=jnp.float32)
    x = x_ref[...] + cat
    o_ref[...] = lax.dot_general(
        w_ref[...], x, (((1,), (1,)), ((), ())),
        preferred_element_type=jnp.float32) + b_ref[...].reshape(NCLS, 1)


def _head_body_aliased(x_ref, i0_ref, i1_ref, i2_ref, cc_ref, w_ref, b_ref,
                       zin_ref, o_ref):
    _head_body_plain(x_ref, i0_ref, i1_ref, i2_ref, cc_ref, w_ref, b_ref,
                     o_ref)


def _make_head(block0, aliased):
    in_specs = [
        pl.BlockSpec((BM, D), lambda i: (i, 0)),
        pl.BlockSpec((1, 1, BM), lambda i: (i, 0, 0)),
        pl.BlockSpec((1, 1, BM), lambda i: (i, 0, 0)),
        pl.BlockSpec((1, 1, BM), lambda i: (i, 0, 0)),
        pl.BlockSpec((3 * NCAT, D), lambda i: (0, 0)),
        pl.BlockSpec((NCLS, D), lambda i: (0, 0)),
        pl.BlockSpec((1, NCLS), lambda i: (0, 0)),
    ]
    kwargs = {}
    body = _head_body_plain
    if aliased:
        in_specs.append(pl.BlockSpec(memory_space=pl.ANY))
        kwargs["input_output_aliases"] = {7: 0}
        body = _head_body_aliased
    return pl.pallas_call(
        body,
        grid=(NBS,),
        in_specs=in_specs,
        out_specs=pl.BlockSpec((NCLS, BM), lambda i: (0, i + block0)),
        out_shape=jax.ShapeDtypeStruct((NCLS, B), jnp.float32),
        **kwargs,
    )


_heads = [_make_head(k * NBS, k > 0) for k in range(NSLICE)]


def kernel(encoded_text, additional_inputs, emb_table, cat_emb_0, cat_emb_1,
           cat_emb_2, fc_w, fc_b):
    text_t = encoded_text.T
    cat_cat = jnp.concatenate([cat_emb_0, cat_emb_1, cat_emb_2], axis=0)
    fc_b2d = fc_b.reshape(1, NCLS)

    def islice(k, s):
        return additional_inputs[k][s * BS:(s + 1) * BS].reshape(NBS, 1, BM)

    xps = [p(text_t, emb_table) for p in _sc_pools]
    zt = _heads[0](xps[0], islice(0, 0), islice(1, 0), islice(2, 0), cat_cat,
                   fc_w, fc_b2d)
    for k in range(1, NSLICE):
        zt = _heads[k](xps[k], islice(0, k), islice(1, k), islice(2, k),
                       cat_cat, fc_w, fc_b2d, zt)
    return zt.T


# speedup vs baseline: 1.0757x; 1.0757x over previous
import functools

import jax
import jax.numpy as jnp
from jax import lax
from jax.experimental import pallas as pl
from jax.experimental.pallas import tpu as pltpu
from jax.experimental.pallas import tpu_sc as plsc

B = 16384
NSLICE = 2
BS = B // NSLICE
L = 20
D = 128
NCLS = 732
NCAT = 100
NC = 2
NS = 16
NW = NC * NS
CB = 8
NJ = D // 16

_F32_MAX = 3.4028235e38


def _tree_sum(xs):
    xs = list(xs)
    while len(xs) > 1:
        nxt = [xs[i] + xs[i + 1] for i in range(0, len(xs) - 1, 2)]
        if len(xs) % 2:
            nxt.append(xs[-1])
        xs = nxt
    return xs[0]


def _make_pool(bs, boff):
    per_w = bs // NW
    chunks = per_w // CB
    npair = chunks // 2

    gd = 2
    gr = CB * L // gd

    def body(textt_hbm, emb_hbm, xpool_hbm, tidxt, tidx_all, rows0, rows1,
             out0, out1, gsem0, gsem1, ssem0, ssem1):
        wid = lax.axis_index("s") * NC + lax.axis_index("c")
        wbase = wid * per_w
        pltpu.sync_copy(textt_hbm.at[:, pl.ds(boff + wbase, per_w)], tidxt)

        lanes = lax.iota(jnp.int32, 16)

        def tgroup(g, _):
            base = g * 16
            for r in range(L):
                v = tidxt[r, pl.ds(base, 16)]
                plsc.store_scatter(tidx_all, [lanes * L + (base * L + r)], v)
            return 0

        lax.fori_loop(0, per_w // 16, tgroup, 0)

        rows = (rows0, rows1)
        outs = (out0, out1)
        gsems = (gsem0, gsem1)
        ssems = (ssem0, ssem1)

        def fire(c, slot):
            for h in range(gd):
                pltpu.async_copy(
                    emb_hbm.at[tidx_all.at[pl.ds(c * CB * L + h * gr, gr)]],
                    rows[slot].at[pl.ds(h * gr, gr)], gsems[slot])

        def wait_gathers(c, slot):
            for h in range(gd):
                pltpu.make_async_copy(
                    emb_hbm.at[tidx_all.at[pl.ds(c * CB * L + h * gr, gr)]],
                    rows[slot].at[pl.ds(h * gr, gr)], gsems[slot]).wait()

        def wait_store(c, slot):
            pltpu.make_async_copy(
                outs[slot], xpool_hbm.at[pl.ds(wbase + c * CB, CB)],
                ssems[slot]).wait()

        def compute(c, slot):
            rows_v = rows[slot]
            out_v = outs[slot]

            def elem_body(e, _):
                ebase = e * L
                acc_a = [jnp.zeros((16,), jnp.float32) for _ in range(NJ)]
                acc_b = [jnp.zeros((16,), jnp.float32) for _ in range(NJ)]
                inds = []
                for r in range(L):
                    vs = [rows_v[ebase + r, pl.ds(j * 16, 16)]
                          for j in range(NJ)]
                    if r % 2 == 0:
                        acc_a = [a + v for a, v in zip(acc_a, vs)]
                    else:
                        acc_b = [a + v for a, v in zip(acc_b, vs)]
                    s = jnp.sum(_tree_sum(vs))
                    inds.append(jnp.where(s != 0.0, 1.0, 0.0))
                cnt = _tree_sum(inds)
                invv = 1.0 / (cnt + jnp.zeros((16,), jnp.float32))
                for j in range(NJ):
                    q = (acc_a[j] + acc_b[j]) * invv
                    q = jnp.where(jnp.isnan(q), 0.0, q)
                    q = jnp.where(q == jnp.inf, _F32_MAX, q)
                    q = jnp.where(q == -jnp.inf, -_F32_MAX, q)
                    out_v[e, pl.ds(j * 16, 16)] = q
                return 0

            lax.fori_loop(0, CB, elem_body, 0)
            pltpu.async_copy(out_v, xpool_hbm.at[pl.ds(wbase + c * CB, CB)],
                             ssems[slot])

        fire(0, 0)
        fire(1, 1)

        def pair_body(p, _):
            for slot in range(2):
                c = 2 * p + slot
                wait_gathers(c, slot)

                @pl.when(p > 0)
                def _():
                    wait_store(c - 2, slot)

                compute(c, slot)

                @pl.when(p < npair - 1)
                def _():
                    fire(c + 2, slot)

            return 0

        lax.fori_loop(0, npair, pair_body, 0)
        wait_store(chunks - 2, 0)
        wait_store(chunks - 1, 1)

    return pl.kernel(
        body,
        out_type=jax.ShapeDtypeStruct((bs, D), jnp.float32),
        mesh=plsc.VectorSubcoreMesh(core_axis_name="c", subcore_axis_name="s"),
        compiler_params=pltpu.CompilerParams(needs_layout_passes=False),
        scratch_types=[
            pltpu.VMEM((L, per_w), jnp.int32),
            pltpu.VMEM((per_w * L,), jnp.int32),
            pltpu.VMEM((CB * L, D), jnp.float32),
            pltpu.VMEM((CB * L, D), jnp.float32),
            pltpu.VMEM((CB, D), jnp.float32),
            pltpu.VMEM((CB, D), jnp.float32),
            pltpu.SemaphoreType.DMA,
            pltpu.SemaphoreType.DMA,
            pltpu.SemaphoreType.DMA,
            pltpu.SemaphoreType.DMA,
        ],
    )


_sc_pools = [_make_pool(BS, k * BS) for k in range(NSLICE)]


BM = 1024
NBS = BS // BM


def _head_body_plain(x_ref, i0_ref, i1_ref, i2_ref, cc_ref, w_ref, b_ref,
                     o_ref):
    span = lax.broadcasted_iota(jnp.int32, (BM, 3 * NCAT), 1)
    oh = ((span == i0_ref[0, 0, :][:, None]).astype(jnp.float32)
          + (span == i1_ref[0, 0, :][:, None] + NCAT).astype(jnp.float32)
          + (span == i2_ref[0, 0, :][:, None] + 2 * NCAT).astype(jnp.float32))
    cat = jnp.dot(oh, cc_ref[...], preferred_element_type=jnp.float32)
    x = x_ref[...] + cat
    o_ref[...] = lax.dot_general(
        w_ref[...], x, (((1,), (1,)), ((), ())),
        preferred_element_type=jnp.float32) + b_ref[...].reshape(NCLS, 1)


def _head_body_aliased(x_ref, i0_ref, i1_ref, i2_ref, cc_ref, w_ref, b_ref,
                       zin_ref, o_ref):
    _head_body_plain(x_ref, i0_ref, i1_ref, i2_ref, cc_ref, w_ref, b_ref,
                     o_ref)


def _make_head(block0, aliased):
    in_specs = [
        pl.BlockSpec((BM, D), lambda i: (i, 0)),
        pl.BlockSpec((1, 1, BM), lambda i: (i, 0, 0)),
        pl.BlockSpec((1, 1, BM), lambda i: (i, 0, 0)),
        pl.BlockSpec((1, 1, BM), lambda i: (i, 0, 0)),
        pl.BlockSpec((3 * NCAT, D), lambda i: (0, 0)),
        pl.BlockSpec((NCLS, D), lambda i: (0, 0)),
        pl.BlockSpec((1, NCLS), lambda i: (0, 0)),
    ]
    kwargs = {}
    body = _head_body_plain
    if aliased:
        in_specs.append(pl.BlockSpec(memory_space=pl.ANY))
        kwargs["input_output_aliases"] = {7: 0}
        body = _head_body_aliased
    return pl.pallas_call(
        body,
        grid=(NBS,),
        in_specs=in_specs,
        out_specs=pl.BlockSpec((NCLS, BM), lambda i: (0, i + block0)),
        out_shape=jax.ShapeDtypeStruct((NCLS, B), jnp.float32),
        **kwargs,
    )


_heads = [_make_head(k * NBS, k > 0) for k in range(NSLICE)]


def kernel(encoded_text, additional_inputs, emb_table, cat_emb_0, cat_emb_1,
           cat_emb_2, fc_w, fc_b):
    text_t = encoded_text.T
    cat_cat = jnp.concatenate([cat_emb_0, cat_emb_1, cat_emb_2], axis=0)
    fc_b2d = fc_b.reshape(1, NCLS)

    def islice(k, s):
        return additional_inputs[k][s * BS:(s + 1) * BS].reshape(NBS, 1, BM)

    xps = [p(text_t, emb_table) for p in _sc_pools]
    zt = _heads[0](xps[0], islice(0, 0), islice(1, 0), islice(2, 0), cat_cat,
                   fc_w, fc_b2d)
    for k in range(1, NSLICE):
        zt = _heads[k](xps[k], islice(0, k), islice(1, k), islice(2, k),
                       cat_cat, fc_w, fc_b2d, zt)
    return zt.T


# speedup vs baseline: 1.0998x; 1.0224x over previous
import functools

import jax
import jax.numpy as jnp
from jax import lax
from jax.experimental import pallas as pl
from jax.experimental.pallas import tpu as pltpu
from jax.experimental.pallas import tpu_sc as plsc

B = 16384
SLICES = [12288, 4096]
OFFS = [0, 12288]
L = 20
D = 128
NCLS = 732
NCAT = 100
NC = 2
NS = 16
NW = NC * NS
CB = 8
NJ = D // 16

_F32_MAX = 3.4028235e38


def _tree_sum(xs):
    xs = list(xs)
    while len(xs) > 1:
        nxt = [xs[i] + xs[i + 1] for i in range(0, len(xs) - 1, 2)]
        if len(xs) % 2:
            nxt.append(xs[-1])
        xs = nxt
    return xs[0]


def _make_pool(bs, boff):
    per_w = bs // NW
    chunks = per_w // CB
    npair = chunks // 2

    gd = 2
    gr = CB * L // gd

    def body(textt_hbm, emb_hbm, xpool_hbm, tidxt, tidx_all, rows0, rows1,
             out0, out1, gsem0, gsem1, ssem0, ssem1):
        wid = lax.axis_index("s") * NC + lax.axis_index("c")
        wbase = wid * per_w
        pltpu.sync_copy(textt_hbm.at[:, pl.ds(boff + wbase, per_w)], tidxt)

        lanes = lax.iota(jnp.int32, 16)

        def tgroup(g, _):
            base = g * 16
            for r in range(L):
                v = tidxt[r, pl.ds(base, 16)]
                plsc.store_scatter(tidx_all, [lanes * L + (base * L + r)], v)
            return 0

        lax.fori_loop(0, per_w // 16, tgroup, 0)

        rows = (rows0, rows1)
        outs = (out0, out1)
        gsems = (gsem0, gsem1)
        ssems = (ssem0, ssem1)

        def fire(c, slot):
            for h in range(gd):
                pltpu.async_copy(
                    emb_hbm.at[tidx_all.at[pl.ds(c * CB * L + h * gr, gr)]],
                    rows[slot].at[pl.ds(h * gr, gr)], gsems[slot])

        def wait_gathers(c, slot):
            for h in range(gd):
                pltpu.make_async_copy(
                    emb_hbm.at[tidx_all.at[pl.ds(c * CB * L + h * gr, gr)]],
                    rows[slot].at[pl.ds(h * gr, gr)], gsems[slot]).wait()

        def wait_store(c, slot):
            pltpu.make_async_copy(
                outs[slot], xpool_hbm.at[pl.ds(wbase + c * CB, CB)],
                ssems[slot]).wait()

        def compute(c, slot):
            rows_v = rows[slot]
            out_v = outs[slot]

            def elem_body(e, _):
                ebase = e * L
                acc_a = [jnp.zeros((16,), jnp.float32) for _ in range(NJ)]
                acc_b = [jnp.zeros((16,), jnp.float32) for _ in range(NJ)]
                inds = []
                for r in range(L):
                    vs = [rows_v[ebase + r, pl.ds(j * 16, 16)]
                          for j in range(NJ)]
                    if r % 2 == 0:
                        acc_a = [a + v for a, v in zip(acc_a, vs)]
                    else:
                        acc_b = [a + v for a, v in zip(acc_b, vs)]
                    s = jnp.sum(_tree_sum(vs))
                    inds.append(jnp.where(s != 0.0, 1.0, 0.0))
                cnt = _tree_sum(inds)
                invv = 1.0 / (cnt + jnp.zeros((16,), jnp.float32))
                for j in range(NJ):
                    q = (acc_a[j] + acc_b[j]) * invv
                    q = jnp.where(jnp.isnan(q), 0.0, q)
                    q = jnp.where(q == jnp.inf, _F32_MAX, q)
                    q = jnp.where(q == -jnp.inf, -_F32_MAX, q)
                    out_v[e, pl.ds(j * 16, 16)] = q
                return 0

            lax.fori_loop(0, CB, elem_body, 0)
            pltpu.async_copy(out_v, xpool_hbm.at[pl.ds(wbase + c * CB, CB)],
                             ssems[slot])

        fire(0, 0)
        fire(1, 1)

        def pair_body(p, _):
            for slot in range(2):
                c = 2 * p + slot
                wait_gathers(c, slot)

                @pl.when(p > 0)
                def _():
                    wait_store(c - 2, slot)

                compute(c, slot)

                @pl.when(p < npair - 1)
                def _():
                    fire(c + 2, slot)

            return 0

        lax.fori_loop(0, npair, pair_body, 0)
        wait_store(chunks - 2, 0)
        wait_store(chunks - 1, 1)

    return pl.kernel(
        body,
        out_type=jax.ShapeDtypeStruct((bs, D), jnp.float32),
        mesh=plsc.VectorSubcoreMesh(core_axis_name="c", subcore_axis_name="s"),
        compiler_params=pltpu.CompilerParams(needs_layout_passes=False),
        scratch_types=[
            pltpu.VMEM((L, per_w), jnp.int32),
            pltpu.VMEM((per_w * L,), jnp.int32),
            pltpu.VMEM((CB * L, D), jnp.float32),
            pltpu.VMEM((CB * L, D), jnp.float32),
            pltpu.VMEM((CB, D), jnp.float32),
            pltpu.VMEM((CB, D), jnp.float32),
            pltpu.SemaphoreType.DMA,
            pltpu.SemaphoreType.DMA,
            pltpu.SemaphoreType.DMA,
            pltpu.SemaphoreType.DMA,
        ],
    )


_sc_pools = [_make_pool(s, o) for s, o in zip(SLICES, OFFS)]


BM = 1024


def _head_body_plain(x_ref, i0_ref, i1_ref, i2_ref, cc_ref, w_ref, b_ref,
                     o_ref):
    span = lax.broadcasted_iota(jnp.int32, (BM, 3 * NCAT), 1)
    oh = ((span == i0_ref[0, 0, :][:, None]).astype(jnp.float32)
          + (span == i1_ref[0, 0, :][:, None] + NCAT).astype(jnp.float32)
          + (span == i2_ref[0, 0, :][:, None] + 2 * NCAT).astype(jnp.float32))
    cat = jnp.dot(oh, cc_ref[...], preferred_element_type=jnp.float32)
    x = x_ref[...] + cat
    o_ref[...] = lax.dot_general(
        w_ref[...], x, (((1,), (1,)), ((), ())),
        preferred_element_type=jnp.float32) + b_ref[...].reshape(NCLS, 1)


def _head_body_aliased(x_ref, i0_ref, i1_ref, i2_ref, cc_ref, w_ref, b_ref,
                       zin_ref, o_ref):
    _head_body_plain(x_ref, i0_ref, i1_ref, i2_ref, cc_ref, w_ref, b_ref,
                     o_ref)


def _make_head(block0, nbs, aliased):
    in_specs = [
        pl.BlockSpec((BM, D), lambda i: (i, 0)),
        pl.BlockSpec((1, 1, BM), lambda i: (i, 0, 0)),
        pl.BlockSpec((1, 1, BM), lambda i: (i, 0, 0)),
        pl.BlockSpec((1, 1, BM), lambda i: (i, 0, 0)),
        pl.BlockSpec((3 * NCAT, D), lambda i: (0, 0)),
        pl.BlockSpec((NCLS, D), lambda i: (0, 0)),
        pl.BlockSpec((1, NCLS), lambda i: (0, 0)),
    ]
    kwargs = {}
    body = _head_body_plain
    if aliased:
        in_specs.append(pl.BlockSpec(memory_space=pl.ANY))
        kwargs["input_output_aliases"] = {7: 0}
        body = _head_body_aliased
    return pl.pallas_call(
        body,
        grid=(nbs,),
        in_specs=in_specs,
        out_specs=pl.BlockSpec((NCLS, BM), lambda i: (0, i + block0)),
        out_shape=jax.ShapeDtypeStruct((NCLS, B), jnp.float32),
        **kwargs,
    )


_heads = [_make_head(o // BM, s // BM, k > 0)
          for k, (s, o) in enumerate(zip(SLICES, OFFS))]


def kernel(encoded_text, additional_inputs, emb_table, cat_emb_0, cat_emb_1,
           cat_emb_2, fc_w, fc_b):
    text_t = encoded_text.T
    cat_cat = jnp.concatenate([cat_emb_0, cat_emb_1, cat_emb_2], axis=0)
    fc_b2d = fc_b.reshape(1, NCLS)

    def islice(k, s):
        return additional_inputs[k][OFFS[s]:OFFS[s] + SLICES[s]].reshape(
            SLICES[s] // BM, 1, BM)

    xps = [p(text_t, emb_table) for p in _sc_pools]
    zt = _heads[0](xps[0], islice(0, 0), islice(1, 0), islice(2, 0), cat_cat,
                   fc_w, fc_b2d)
    for k in range(1, len(SLICES)):
        zt = _heads[k](xps[k], islice(0, k), islice(1, k), islice(2, k),
                       cat_cat, fc_w, fc_b2d, zt)
    return zt.T
